# baseline (speedup 1.0000x reference)
import jax
import jax.numpy as jnp
from jax.experimental import pallas as pl
from jax.experimental.pallas import tpu as pltpu

_BATCH = 16
_N_PATCH = 64
_D = 128
_N_MEM = 100000
_Q = _BATCH * _N_PATCH

_BLK = 2000
_NBLK = _N_MEM // _BLK


def _prep(mf, mb_ref, msq_ref):
    msq_ref[...] = 0.5 * jnp.sum(mf * mf, axis=1, keepdims=True)
    mb_ref[...] = mf.astype(jnp.bfloat16)


def _mm_min(mb_ref, msq_ref, qb_ref):
    p = jax.lax.dot_general(mb_ref[...], qb_ref[...],
                            (((1,), (1,)), ((), ())),
                            preferred_element_type=jnp.float32)
    return jnp.min(msq_ref[...] - p, axis=0, keepdims=True)


def _knn_kernel(m0_ref, mn_ref, q_ref, out_ref, min_ref,
                acc_ref, qb_ref, qs_ref, mba_ref, mbb_ref,
                msqa_ref, msqb_ref):
    i = pl.program_id(0)
    even = jax.lax.rem(i, 2) == 0

    @pl.when(i == 0)
    def _():
        qf = q_ref[...]
        qb_ref[...] = qf.astype(jnp.bfloat16)
        qsq = (qf * qf).astype(jnp.bfloat16)
        ones = jnp.ones((1, _D), jnp.bfloat16)
        qs_ref[...] = jax.lax.dot_general(
            ones, qsq, (((1,), (1,)), ((), ())),
            preferred_element_type=jnp.float32)
        _prep(m0_ref[...], mba_ref, msqa_ref)

    @pl.when(even)
    def _():
        @pl.when(i < _NBLK - 1)
        def _():
            _prep(mn_ref[...], mbb_ref, msqb_ref)
        min_ref[...] = _mm_min(mba_ref, msqa_ref, qb_ref)

    @pl.when(jnp.logical_not(even))
    def _():
        @pl.when(i < _NBLK - 1)
        def _():
            _prep(mn_ref[...], mba_ref, msqa_ref)
        min_ref[...] = _mm_min(mbb_ref, msqb_ref, qb_ref)

    bmin = min_ref[...]

    @pl.when(i == 0)
    def _():
        acc_ref[...] = bmin

    @pl.when(i > 0)
    def _():
        acc_ref[...] = jnp.minimum(acc_ref[...], bmin)

    @pl.when(i == _NBLK - 1)
    def _():
        d2 = jnp.maximum(2.0 * acc_ref[...] + qs_ref[...], 0.0) + 1e-12
        v = d2
        for k in (1, 2, 4, 8, 16, 32):
            v = jnp.maximum(v, pltpu.roll(v, _Q - k, axis=1))
        out_ref[...] = jnp.sqrt(v)


def kernel(queries, memory_bank):
    v = pl.pallas_call(
        _knn_kernel,
        grid=(_NBLK,),
        in_specs=[
            pl.BlockSpec((_BLK, _D), lambda i: (0, 0)),
            pl.BlockSpec((_BLK, _D),
                         lambda i: (jnp.minimum(i + 1, _NBLK - 1), 0)),
            pl.BlockSpec((_Q, _D), lambda i: (0, 0)),
        ],
        out_specs=pl.BlockSpec((1, _Q), lambda i: (0, 0)),
        out_shape=jax.ShapeDtypeStruct((1, _Q), jnp.float32),
        scratch_shapes=[
            pltpu.VMEM((1, _Q), jnp.float32),
            pltpu.VMEM((1, _Q), jnp.float32),
            pltpu.VMEM((_Q, _D), jnp.bfloat16),
            pltpu.VMEM((1, _Q), jnp.float32),
            pltpu.VMEM((_BLK, _D), jnp.bfloat16),
            pltpu.VMEM((_BLK, _D), jnp.bfloat16),
            pltpu.VMEM((_BLK, 1), jnp.float32),
            pltpu.VMEM((_BLK, 1), jnp.float32),
        ],
    )(memory_bank, memory_bank, queries)

    return v.reshape(_BATCH, _N_PATCH)[:, 0]


# speedup vs baseline: 1.2439x; 1.2439x over previous
import jax
import jax.numpy as jnp
from jax.experimental import pallas as pl
from jax.experimental.pallas import tpu as pltpu

_BATCH = 16
_N_PATCH = 64
_D = 128
_N_MEM = 100000
_Q = _BATCH * _N_PATCH

_BLK = 2000
_NBLK = _N_MEM // _BLK


def _knn_kernel(m_ref, q_ref, out_ref, acc_ref, qb_ref, qs_ref):
    i = pl.program_id(0)

    @pl.when(i == 0)
    def _():
        qf = q_ref[...]
        qb_ref[...] = qf.astype(jnp.bfloat16)
        qsq = (qf * qf).astype(jnp.bfloat16)
        ones = jnp.ones((1, _D), jnp.bfloat16)
        qs_ref[...] = jax.lax.dot_general(
            ones, qsq, (((1,), (1,)), ((), ())),
            preferred_element_type=jnp.float32)

    mf = m_ref[...]
    msq_half = 0.5 * jnp.sum(mf * mf, axis=1, keepdims=True)
    m = mf.astype(jnp.bfloat16)
    p = jax.lax.dot_general(m, qb_ref[...], (((1,), (1,)), ((), ())),
                            preferred_element_type=jnp.float32)
    s = msq_half - p
    bmin = jnp.min(s, axis=0, keepdims=True)

    @pl.when(i == 0)
    def _():
        acc_ref[...] = bmin

    @pl.when(i > 0)
    def _():
        acc_ref[...] = jnp.minimum(acc_ref[...], bmin)

    @pl.when(i == _NBLK - 1)
    def _():
        d2 = jnp.maximum(2.0 * acc_ref[...] + qs_ref[...], 0.0) + 1e-12
        v = d2
        for k in (1, 2, 4, 8, 16, 32):
            v = jnp.maximum(v, pltpu.roll(v, _Q - k, axis=1))
        out_ref[...] = jnp.sqrt(v)


def kernel(queries, memory_bank):
    v = pl.pallas_call(
        _knn_kernel,
        grid=(_NBLK,),
        in_specs=[
            pl.BlockSpec((_BLK, _D), lambda i: (i, 0)),
            pl.BlockSpec((_Q, _D), lambda i: (0, 0)),
        ],
        out_specs=pl.BlockSpec((1, _Q), lambda i: (0, 0)),
        out_shape=jax.ShapeDtypeStruct((1, _Q), jnp.float32),
        scratch_shapes=[
            pltpu.VMEM((1, _Q), jnp.float32),
            pltpu.VMEM((_Q, _D), jnp.bfloat16),
            pltpu.VMEM((1, _Q), jnp.float32),
        ],
    )(memory_bank, queries)

    return v.reshape(_BATCH, _N_PATCH)[:, 0]
